# baseline (speedup 1.0000x reference)
import functools

import jax
import jax.numpy as jnp
from jax import lax
from jax.experimental import pallas as pl
from jax.experimental.pallas import tpu as pltpu
from jax.experimental.pallas import tpu_sc as plsc

N = 10000
E = 320000
F_IN = 128
D = 64
NG = 64
G3 = 3 * D

NC = 2
NS = 16
NW = NC * NS
EW = E // NW
CK = 80
NCHUNK = EW // CK
RA = 624
RL = 640
DEGW = 16

_f32 = jnp.float32
_HI = lax.Precision.HIGHEST


def _mesh():
    return plsc.VectorSubcoreMesh(core_axis_name="c", subcore_axis_name="s")


_CP = pltpu.CompilerParams(needs_layout_passes=False,
                           use_tc_tiling_on_sc=False)


def _gather_body(out_hbm, src_hbm, dst_hbm, ea0_hbm, ea1_hbm, g_hbm,
                 src_v, dst_v, ea0_v, ea1_v, xi_rows, xj_rows, g_buf,
                 sem_i, sem_j):
    c = lax.axis_index("c")
    s = lax.axis_index("s")
    wid = s * NC + c
    wbase = wid * EW

    def chunk(i, carry):
        base = wbase + i * CK
        pltpu.sync_copy(src_hbm.at[pl.ds(base, CK)], src_v)
        pltpu.sync_copy(dst_hbm.at[pl.ds(base, CK)], dst_v)
        pltpu.sync_copy(ea0_hbm.at[pl.ds(base, CK)], ea0_v)
        pltpu.sync_copy(ea1_hbm.at[pl.ds(base, CK)], ea1_v)
        ci = pltpu.async_copy(out_hbm.at[dst_v], xi_rows, sem_i)
        cj = pltpu.async_copy(out_hbm.at[src_v], xj_rows, sem_j)
        ci.wait()
        cj.wait()

        def group(g, carry2):
            e0 = g * 16
            p0v = ea0_v[pl.ds(e0, 16)]
            p1v = ea1_v[pl.ds(e0, 16)]
            for lane in range(16):
                e = e0 + lane
                p0 = p0v[lane]
                p1 = p1v[lane]
                for j in range(D // 16):
                    xj = xj_rows[e, pl.ds(j * 16, 16)]
                    g_buf[e, pl.ds(j * 16, 16)] = xi_rows[e, pl.ds(j * 16, 16)]
                    g_buf[e, pl.ds(D + j * 16, 16)] = p0 * xj
                    g_buf[e, pl.ds(2 * D + j * 16, 16)] = p1 * xj
            return carry2

        lax.fori_loop(0, CK // 16, group, 0)
        pltpu.sync_copy(g_buf, g_hbm.at[pl.ds(base, CK)])
        return carry

    lax.fori_loop(0, NCHUNK, chunk, 0)


@functools.cache
def _get_gather():
    return pl.kernel(
        _gather_body,
        out_type=jax.ShapeDtypeStruct((E, G3), _f32),
        mesh=_mesh(),
        compiler_params=_CP,
        scratch_types=[
            pltpu.VMEM((CK,), jnp.int32),
            pltpu.VMEM((CK,), jnp.int32),
            pltpu.VMEM((CK,), _f32),
            pltpu.VMEM((CK,), _f32),
            pltpu.VMEM((CK, D), _f32),
            pltpu.VMEM((CK, D), _f32),
            pltpu.VMEM((CK, G3), _f32),
            pltpu.SemaphoreType.DMA,
            pltpu.SemaphoreType.DMA,
        ],
    )


EH = E // 2
SCAN_B = 2000
NBATCH = EH // SCAN_B
CAP = 2016
GB = 80
RJ = RL + 16
JUNK = RL


def _make_scatter(with_deg):
    def body(u_hbm, dst_hbm, *rest):
        if with_deg:
            (acc_out, deg_out, acc_l, deg_l, dstb0, dstb1, prow_l, dloc_l,
             ur0, ur1, semd0, semd1, semu0, semu1) = rest
        else:
            (acc_out, acc_l, dstb0, dstb1, prow_l, dloc_l,
             ur0, ur1, semd0, semd1, semu0, semu1) = rest
        c = lax.axis_index("c")
        s = lax.axis_index("s")
        lo = s * RA
        mysz = jnp.where(s == NS - 1, RL, RA)
        hi = lo + mysz
        ebase = c * EH
        io16 = lax.iota(jnp.int32, 16)

        def zacc(r, cr):
            for j in range(D // 16):
                acc_l[r, pl.ds(j * 16, 16)] = jnp.zeros((16,), _f32)
            if with_deg:
                deg_l[r, pl.ds(0, 16)] = jnp.zeros((16,), _f32)
            return cr

        lax.fori_loop(0, RJ, zacc, 0)

        def zlist(q, cr):
            prow_l[pl.ds(q * 16, 16)] = jnp.zeros((16,), jnp.int32)
            return cr

        lax.fori_loop(0, CAP // 16, zlist, 0)

        def start_dst(bi, buf, sem):
            pltpu.async_copy(
                dst_hbm.at[pl.ds(ebase + bi * SCAN_B, SCAN_B)], buf, sem)

        def wait_dst(buf, sem):
            pltpu.make_async_copy(
                dst_hbm.at[pl.ds(ebase, SCAN_B)], buf, sem).wait()

        def start_u(b, ur, semu):
            pltpu.async_copy(u_hbm.at[prow_l.at[pl.ds(b * GB, GB)]],
                             ur, semu)

        def wait_u(ur, semu):
            pltpu.make_async_copy(u_hbm.at[pl.ds(0, GB)], ur, semu).wait()

        start_dst(0, dstb0, semd0)
        start_dst(1, dstb1, semd1)

        def half_batch(bi, dstb, semd):
            wait_dst(dstb, semd)

            def scan_g(g, off):
                dv = dstb[pl.ds(g * 16, 16)]
                m = (dv >= lo) & (dv < hi)
                csum = plsc.cumsum(m.astype(jnp.int32))
                idxs = off + csum - 1
                posv = ebase + bi * SCAN_B + g * 16 + io16
                plsc.store_scatter(prow_l, [idxs], posv, mask=m)
                plsc.store_scatter(dloc_l, [idxs], dv - lo, mask=m)
                return off + csum[15]

            k = lax.fori_loop(0, SCAN_B // 16, scan_g, 0)

            @pl.when(bi + 2 < NBATCH)
            def _pf():
                start_dst(bi + 2, dstb, semd)

            padidx = k + io16
            plsc.store_scatter(dloc_l, [padidx],
                               jnp.full((16,), JUNK, jnp.int32))
            plsc.store_scatter(prow_l, [padidx], jnp.zeros((16,), jnp.int32))
            nb = (k + GB - 1) // GB

            @pl.when(nb > 0)
            def _s0():
                start_u(0, ur0, semu0)

            @pl.when(nb > 1)
            def _s1():
                start_u(1, ur1, semu1)

            def process(b, ur):
                kb = k - b * GB
                ng = jnp.minimum(GB // 16, (kb + 15) >> 4)

                def grp(gl, cr3):
                    qb = b * GB + gl * 16
                    rv = dloc_l[pl.ds(qb, 16)]
                    for lane in range(16):
                        r = rv[lane]
                        for j in range(D // 16):
                            acc_l[r, pl.ds(j * 16, 16)] = (
                                acc_l[r, pl.ds(j * 16, 16)]
                                + ur[gl * 16 + lane, pl.ds(j * 16, 16)])
                        if with_deg:
                            deg_l[r, pl.ds(0, 16)] = (
                                deg_l[r, pl.ds(0, 16)]
                                + jnp.ones((16,), _f32))
                    return cr3

                lax.fori_loop(0, ng, grp, 0)

            def bpair(bp, cr2):
                b0 = 2 * bp
                b1 = 2 * bp + 1

                @pl.when(b0 < nb)
                def _p0():
                    wait_u(ur0, semu0)

                    @pl.when(b0 + 2 < nb)
                    def _n0():
                        start_u(b0 + 2, ur0, semu0)

                    process(b0, ur0)

                @pl.when(b1 < nb)
                def _p1():
                    wait_u(ur1, semu1)

                    @pl.when(b1 + 2 < nb)
                    def _n1():
                        start_u(b1 + 2, ur1, semu1)

                    process(b1, ur1)
                return cr2

            lax.fori_loop(0, (nb + 1) >> 1, bpair, 0)

        def batch_pair(q, cr):
            half_batch(2 * q, dstb0, semd0)
            half_batch(2 * q + 1, dstb1, semd1)
            return cr

        lax.fori_loop(0, NBATCH // 2, batch_pair, 0)

        @pl.when(s < NS - 1)
        def _dump_small():
            pltpu.sync_copy(acc_l.at[pl.ds(0, RA)],
                            acc_out.at[c, pl.ds(lo, RA)])
            if with_deg:
                pltpu.sync_copy(deg_l.at[pl.ds(0, RA)],
                                deg_out.at[c, pl.ds(lo, RA)])

        @pl.when(s == NS - 1)
        def _dump_last():
            pltpu.sync_copy(acc_l.at[pl.ds(0, RL)],
                            acc_out.at[c, pl.ds((NS - 1) * RA, RL)])
            if with_deg:
                pltpu.sync_copy(deg_l.at[pl.ds(0, RL)],
                                deg_out.at[c, pl.ds((NS - 1) * RA, RL)])

    scratch = [
        pltpu.VMEM((RJ, D), _f32),
        pltpu.VMEM((RJ, DEGW), _f32),
        pltpu.VMEM((SCAN_B,), jnp.int32),
        pltpu.VMEM((SCAN_B,), jnp.int32),
        pltpu.VMEM((CAP,), jnp.int32),
        pltpu.VMEM((CAP,), jnp.int32),
        pltpu.VMEM((GB, D), _f32),
        pltpu.VMEM((GB, D), _f32),
        pltpu.SemaphoreType.DMA,
        pltpu.SemaphoreType.DMA,
        pltpu.SemaphoreType.DMA,
        pltpu.SemaphoreType.DMA,
    ]
    if with_deg:
        out_type = (jax.ShapeDtypeStruct((NC, N, D), _f32),
                    jax.ShapeDtypeStruct((NC, N, DEGW), _f32))
    else:
        out_type = jax.ShapeDtypeStruct((NC, N, D), _f32)
        scratch = scratch[:1] + scratch[2:]
    return pl.kernel(
        body, out_type=out_type, mesh=_mesh(),
        compiler_params=_CP,
        scratch_types=scratch)


_get_scatter = functools.cache(_make_scatter)


def _d0_body(x_ref, wpre_ref, bpre_ref, out_ref):
    out_ref[...] = jnp.maximum(x_ref[...] @ wpre_ref[...] + bpre_ref[...],
                               0.0)


def _d0(x, wpre, bpre):
    return pl.pallas_call(
        _d0_body,
        out_shape=jax.ShapeDtypeStruct((N, D), _f32),
    )(x, wpre, bpre)


def _norm_agg(acc_ref, deg_ref, g_ref, bb_ref):
    deg = jnp.maximum(deg_ref[0, :, 0:1] + deg_ref[1, :, 0:1], 1.0)
    agg = (acc_ref[0] + acc_ref[1]) / deg
    mu = jnp.mean(agg, axis=0, keepdims=True)
    xc = agg - mu
    var = jnp.mean(xc * xc, axis=0, keepdims=True)
    return g_ref[...] * xc * lax.rsqrt(var + 1e-5) + bb_ref[...]


def _dmid_body(acc_ref, deg_ref, g_ref, bb_ref, out_ref):
    out_ref[...] = _norm_agg(acc_ref, deg_ref, g_ref, bb_ref)


def _dmid(acc, degt, g, bb):
    return pl.pallas_call(
        _dmid_body,
        out_shape=jax.ShapeDtypeStruct((N, D), _f32),
    )(acc, degt, g, bb)


def _df_body(acc_ref, deg_ref, g_ref, bb_ref, batch_ref, wpost_ref, bpost_ref,
             wout_ref, bout_ref, o_ref):
    outn = _norm_agg(acc_ref, deg_ref, g_ref, bb_ref)
    ids = lax.broadcasted_iota(jnp.int32, (NG, N), 0)
    oh = (batch_ref[...] == ids).astype(_f32)
    psum = jnp.dot(oh, outn, precision=_HI)
    cnt = jnp.maximum(jnp.sum(oh, axis=1, keepdims=True), 1.0)
    pooled = psum / cnt
    h = jnp.maximum(pooled @ wpost_ref[...] + bpost_ref[...], 0.0)
    o_ref[...] = h @ wout_ref[...] + bout_ref[...]


def _df(acc, degt, g, bb, batch2, wpost, bpost, wout, bout):
    return pl.pallas_call(
        _df_body,
        out_shape=jax.ShapeDtypeStruct((NG, 1), _f32),
    )(acc, degt, g, bb, batch2, wpost, bpost, wout, bout)


EB = 2000


def _m_body(g_ref, w0_ref, b0_ref, w1_ref, b1_ref, u_ref):
    m = jnp.maximum(g_ref[...] @ w0_ref[...] + b0_ref[...], 0.0)
    u_ref[...] = jnp.maximum(m @ w1_ref[...] + b1_ref[...], 0.0)


def _m(g, w0, b0, w1, b1):
    return pl.pallas_call(
        _m_body,
        grid=(E // EB,),
        in_specs=[
            pl.BlockSpec((EB, G3), lambda i: (i, 0)),
            pl.BlockSpec((G3, D), lambda i: (0, 0)),
            pl.BlockSpec((1, D), lambda i: (0, 0)),
            pl.BlockSpec((D, D), lambda i: (0, 0)),
            pl.BlockSpec((1, D), lambda i: (0, 0)),
        ],
        out_specs=pl.BlockSpec((EB, D), lambda i: (i, 0)),
        out_shape=jax.ShapeDtypeStruct((E, D), _f32),
    )(g, w0, b0, w1, b1)


def kernel(x, edge_index, edge_attr, batch, W_pre, b_pre,
           conv0_W0, conv0_b0, conv0_W1, conv0_b1, bn0_g, bn0_b,
           conv1_W0, conv1_b0, conv1_W1, conv1_b1, bn1_g, bn1_b,
           conv2_W0, conv2_b0, conv2_W1, conv2_b1, bn2_g, bn2_b,
           W_post, b_post, W_out, b_out):
    src = edge_index[0]
    dst = edge_index[1]
    ea0 = jnp.asarray(edge_attr[:, 0])
    ea1 = jnp.asarray(edge_attr[:, 1])
    batch2 = batch.reshape(1, N)

    convs = [(conv0_W0, conv0_b0, conv0_W1, conv0_b1, bn0_g, bn0_b),
             (conv1_W0, conv1_b0, conv1_W1, conv1_b1, bn1_g, bn1_b),
             (conv2_W0, conv2_b0, conv2_W1, conv2_b1, bn2_g, bn2_b)]

    out = _d0(x, W_pre, b_pre.reshape(1, D))
    degt = None
    acc = None
    for i, (W0, b0, W1, b1, g, bb) in enumerate(convs):
        gmat = _get_gather()(out, src, dst, ea0, ea1)
        u = _m(gmat, W0, b0.reshape(1, D), W1, b1.reshape(1, D))
        if i == 0:
            acc, degt = _get_scatter(True)(u, dst)
        else:
            acc = _get_scatter(False)(u, dst)
        if i < 2:
            out = _dmid(acc, degt, g.reshape(1, D), bb.reshape(1, D))
    o = _df(acc, degt, bn2_g.reshape(1, D), bn2_b.reshape(1, D), batch2,
            W_post, b_post.reshape(1, D), W_out, b_out.reshape(1, 1))
    return o.reshape(-1)


# speedup vs baseline: 1.0525x; 1.0525x over previous
import functools

import jax
import jax.numpy as jnp
from jax import lax
from jax.experimental import pallas as pl
from jax.experimental.pallas import tpu as pltpu
from jax.experimental.pallas import tpu_sc as plsc

N = 10000
E = 320000
F_IN = 128
D = 64
NG = 64
G3 = 3 * D

NC = 2
NS = 16
NW = NC * NS
EW = E // NW
CK = 80
NCHUNK = EW // CK
RA = 624
RL = 640
DEGW = 16

_f32 = jnp.float32
_HI = lax.Precision.HIGHEST


def _mesh():
    return plsc.VectorSubcoreMesh(core_axis_name="c", subcore_axis_name="s")


_CP = pltpu.CompilerParams(needs_layout_passes=False,
                           use_tc_tiling_on_sc=False)


def _gather_body(out_hbm, src_hbm, dst_hbm, ea0_hbm, ea1_hbm, g_hbm,
                 src_v, dst_v, ea0_v, ea1_v, xi_rows, xj_rows, g_buf,
                 sem_i, sem_j, sem_x):
    c = lax.axis_index("c")
    s = lax.axis_index("s")
    wid = s * NC + c
    wbase = wid * EW

    def chunk(i, carry):
        base = wbase + i * CK
        c1 = pltpu.async_copy(src_hbm.at[pl.ds(base, CK)], src_v, sem_i)
        c2 = pltpu.async_copy(dst_hbm.at[pl.ds(base, CK)], dst_v, sem_j)
        c3 = pltpu.async_copy(ea0_hbm.at[pl.ds(base, CK)], ea0_v, sem_x)
        c4 = pltpu.async_copy(ea1_hbm.at[pl.ds(base, CK)], ea1_v, sem_x)
        c1.wait()
        c2.wait()
        ci = pltpu.async_copy(out_hbm.at[dst_v], xi_rows, sem_i)
        cj = pltpu.async_copy(out_hbm.at[src_v], xj_rows, sem_j)
        c3.wait()
        c4.wait()
        ci.wait()
        cj.wait()

        def group(g, carry2):
            e0 = g * 16
            p0v = ea0_v[pl.ds(e0, 16)]
            p1v = ea1_v[pl.ds(e0, 16)]
            for lane in range(16):
                e = e0 + lane
                p0 = p0v[lane]
                p1 = p1v[lane]
                for j in range(D // 16):
                    xj = xj_rows[e, pl.ds(j * 16, 16)]
                    g_buf[e, pl.ds(j * 16, 16)] = xi_rows[e, pl.ds(j * 16, 16)]
                    g_buf[e, pl.ds(D + j * 16, 16)] = p0 * xj
                    g_buf[e, pl.ds(2 * D + j * 16, 16)] = p1 * xj
            return carry2

        lax.fori_loop(0, CK // 16, group, 0)
        pltpu.sync_copy(g_buf, g_hbm.at[pl.ds(base, CK)])
        return carry

    lax.fori_loop(0, NCHUNK, chunk, 0)


@functools.cache
def _get_gather():
    return pl.kernel(
        _gather_body,
        out_type=jax.ShapeDtypeStruct((E, G3), _f32),
        mesh=_mesh(),
        compiler_params=_CP,
        scratch_types=[
            pltpu.VMEM((CK,), jnp.int32),
            pltpu.VMEM((CK,), jnp.int32),
            pltpu.VMEM((CK,), _f32),
            pltpu.VMEM((CK,), _f32),
            pltpu.VMEM((CK, D), _f32),
            pltpu.VMEM((CK, D), _f32),
            pltpu.VMEM((CK, G3), _f32),
            pltpu.SemaphoreType.DMA,
            pltpu.SemaphoreType.DMA,
            pltpu.SemaphoreType.DMA,
        ],
    )


EH = E // 2
SCAN_B = 2000
NBATCH = EH // SCAN_B
CAP = 2016
GB = 80
RJ = RL + 16
JUNK = RL


def _make_scatter(with_deg):
    def body(u_hbm, dst_hbm, *rest):
        if with_deg:
            (acc_out, deg_out, acc_l, deg_l, dstb0, dstb1, prow_l, dloc_l,
             ur0, ur1, semd0, semd1, semu0, semu1) = rest
        else:
            (acc_out, acc_l, dstb0, dstb1, prow_l, dloc_l,
             ur0, ur1, semd0, semd1, semu0, semu1) = rest
        c = lax.axis_index("c")
        s = lax.axis_index("s")
        lo = s * RA
        mysz = jnp.where(s == NS - 1, RL, RA)
        hi = lo + mysz
        ebase = c * EH
        io16 = lax.iota(jnp.int32, 16)

        def zacc(r, cr):
            for j in range(D // 16):
                acc_l[r, pl.ds(j * 16, 16)] = jnp.zeros((16,), _f32)
            if with_deg:
                deg_l[r, pl.ds(0, 16)] = jnp.zeros((16,), _f32)
            return cr

        lax.fori_loop(0, RJ, zacc, 0)

        def zlist(q, cr):
            prow_l[pl.ds(q * 16, 16)] = jnp.zeros((16,), jnp.int32)
            return cr

        lax.fori_loop(0, CAP // 16, zlist, 0)

        def start_dst(bi, buf, sem):
            pltpu.async_copy(
                dst_hbm.at[pl.ds(ebase + bi * SCAN_B, SCAN_B)], buf, sem)

        def wait_dst(buf, sem):
            pltpu.make_async_copy(
                dst_hbm.at[pl.ds(ebase, SCAN_B)], buf, sem).wait()

        def start_u(b, ur, semu):
            pltpu.async_copy(u_hbm.at[prow_l.at[pl.ds(b * GB, GB)]],
                             ur, semu)

        def wait_u(ur, semu):
            pltpu.make_async_copy(u_hbm.at[pl.ds(0, GB)], ur, semu).wait()

        start_dst(0, dstb0, semd0)
        start_dst(1, dstb1, semd1)

        def half_batch(bi, dstb, semd):
            wait_dst(dstb, semd)

            def scan_g(g, off):
                dv = dstb[pl.ds(g * 16, 16)]
                m = (dv >= lo) & (dv < hi)
                csum = plsc.cumsum(m.astype(jnp.int32))
                idxs = off + csum - 1
                posv = ebase + bi * SCAN_B + g * 16 + io16
                plsc.store_scatter(prow_l, [idxs], posv, mask=m)
                plsc.store_scatter(dloc_l, [idxs], dv - lo, mask=m)
                return off + csum[15]

            k = lax.fori_loop(0, SCAN_B // 16, scan_g, 0,
                              unroll=4)

            @pl.when(bi + 2 < NBATCH)
            def _pf():
                start_dst(bi + 2, dstb, semd)

            padidx = k + io16
            plsc.store_scatter(dloc_l, [padidx],
                               jnp.full((16,), JUNK, jnp.int32))
            plsc.store_scatter(prow_l, [padidx], jnp.zeros((16,), jnp.int32))
            nb = (k + GB - 1) // GB

            @pl.when(nb > 0)
            def _s0():
                start_u(0, ur0, semu0)

            @pl.when(nb > 1)
            def _s1():
                start_u(1, ur1, semu1)

            def process(b, ur):
                kb = k - b * GB
                ng = jnp.minimum(GB // 16, (kb + 15) >> 4)

                def grp(gl, cr3):
                    qb = b * GB + gl * 16
                    rv = dloc_l[pl.ds(qb, 16)]
                    for lane in range(16):
                        r = rv[lane]
                        for j in range(D // 16):
                            acc_l[r, pl.ds(j * 16, 16)] = (
                                acc_l[r, pl.ds(j * 16, 16)]
                                + ur[gl * 16 + lane, pl.ds(j * 16, 16)])
                        if with_deg:
                            deg_l[r, pl.ds(0, 16)] = (
                                deg_l[r, pl.ds(0, 16)]
                                + jnp.ones((16,), _f32))
                    return cr3

                lax.fori_loop(0, ng, grp, 0)

            def bpair(bp, cr2):
                b0 = 2 * bp
                b1 = 2 * bp + 1

                @pl.when(b0 < nb)
                def _p0():
                    wait_u(ur0, semu0)

                    @pl.when(b0 + 2 < nb)
                    def _n0():
                        start_u(b0 + 2, ur0, semu0)

                    process(b0, ur0)

                @pl.when(b1 < nb)
                def _p1():
                    wait_u(ur1, semu1)

                    @pl.when(b1 + 2 < nb)
                    def _n1():
                        start_u(b1 + 2, ur1, semu1)

                    process(b1, ur1)
                return cr2

            lax.fori_loop(0, (nb + 1) >> 1, bpair, 0)

        def batch_pair(q, cr):
            half_batch(2 * q, dstb0, semd0)
            half_batch(2 * q + 1, dstb1, semd1)
            return cr

        lax.fori_loop(0, NBATCH // 2, batch_pair, 0)

        @pl.when(s < NS - 1)
        def _dump_small():
            pltpu.sync_copy(acc_l.at[pl.ds(0, RA)],
                            acc_out.at[c, pl.ds(lo, RA)])
            if with_deg:
                pltpu.sync_copy(deg_l.at[pl.ds(0, RA)],
                                deg_out.at[c, pl.ds(lo, RA)])

        @pl.when(s == NS - 1)
        def _dump_last():
            pltpu.sync_copy(acc_l.at[pl.ds(0, RL)],
                            acc_out.at[c, pl.ds((NS - 1) * RA, RL)])
            if with_deg:
                pltpu.sync_copy(deg_l.at[pl.ds(0, RL)],
                                deg_out.at[c, pl.ds((NS - 1) * RA, RL)])

    scratch = [
        pltpu.VMEM((RJ, D), _f32),
        pltpu.VMEM((RJ, DEGW), _f32),
        pltpu.VMEM((SCAN_B,), jnp.int32),
        pltpu.VMEM((SCAN_B,), jnp.int32),
        pltpu.VMEM((CAP,), jnp.int32),
        pltpu.VMEM((CAP,), jnp.int32),
        pltpu.VMEM((GB, D), _f32),
        pltpu.VMEM((GB, D), _f32),
        pltpu.SemaphoreType.DMA,
        pltpu.SemaphoreType.DMA,
        pltpu.SemaphoreType.DMA,
        pltpu.SemaphoreType.DMA,
    ]
    if with_deg:
        out_type = (jax.ShapeDtypeStruct((NC, N, D), _f32),
                    jax.ShapeDtypeStruct((NC, N, DEGW), _f32))
    else:
        out_type = jax.ShapeDtypeStruct((NC, N, D), _f32)
        scratch = scratch[:1] + scratch[2:]
    return pl.kernel(
        body, out_type=out_type, mesh=_mesh(),
        compiler_params=_CP,
        scratch_types=scratch)


_get_scatter = functools.cache(_make_scatter)


def _d0_body(x_ref, wpre_ref, bpre_ref, out_ref):
    out_ref[...] = jnp.maximum(x_ref[...] @ wpre_ref[...] + bpre_ref[...],
                               0.0)


def _d0(x, wpre, bpre):
    return pl.pallas_call(
        _d0_body,
        out_shape=jax.ShapeDtypeStruct((N, D), _f32),
    )(x, wpre, bpre)


def _norm_agg(acc_ref, deg_ref, g_ref, bb_ref):
    deg = jnp.maximum(deg_ref[0, :, 0:1] + deg_ref[1, :, 0:1], 1.0)
    agg = (acc_ref[0] + acc_ref[1]) / deg
    mu = jnp.mean(agg, axis=0, keepdims=True)
    xc = agg - mu
    var = jnp.mean(xc * xc, axis=0, keepdims=True)
    return g_ref[...] * xc * lax.rsqrt(var + 1e-5) + bb_ref[...]


def _dmid_body(acc_ref, deg_ref, g_ref, bb_ref, out_ref):
    out_ref[...] = _norm_agg(acc_ref, deg_ref, g_ref, bb_ref)


def _dmid(acc, degt, g, bb):
    return pl.pallas_call(
        _dmid_body,
        out_shape=jax.ShapeDtypeStruct((N, D), _f32),
    )(acc, degt, g, bb)


def _df_body(acc_ref, deg_ref, g_ref, bb_ref, batch_ref, wpost_ref, bpost_ref,
             wout_ref, bout_ref, o_ref):
    outn = _norm_agg(acc_ref, deg_ref, g_ref, bb_ref)
    ids = lax.broadcasted_iota(jnp.int32, (NG, N), 0)
    oh = (batch_ref[...] == ids).astype(_f32)
    psum = jnp.dot(oh, outn, precision=_HI)
    cnt = jnp.maximum(jnp.sum(oh, axis=1, keepdims=True), 1.0)
    pooled = psum / cnt
    h = jnp.maximum(pooled @ wpost_ref[...] + bpost_ref[...], 0.0)
    o_ref[...] = h @ wout_ref[...] + bout_ref[...]


def _df(acc, degt, g, bb, batch2, wpost, bpost, wout, bout):
    return pl.pallas_call(
        _df_body,
        out_shape=jax.ShapeDtypeStruct((NG, 1), _f32),
    )(acc, degt, g, bb, batch2, wpost, bpost, wout, bout)


EB = 2000


def _m_body(g_ref, w0_ref, b0_ref, w1_ref, b1_ref, u_ref):
    m = jnp.maximum(g_ref[...] @ w0_ref[...] + b0_ref[...], 0.0)
    u_ref[...] = jnp.maximum(m @ w1_ref[...] + b1_ref[...], 0.0)


def _m(g, w0, b0, w1, b1):
    return pl.pallas_call(
        _m_body,
        grid=(E // EB,),
        in_specs=[
            pl.BlockSpec((EB, G3), lambda i: (i, 0)),
            pl.BlockSpec((G3, D), lambda i: (0, 0)),
            pl.BlockSpec((1, D), lambda i: (0, 0)),
            pl.BlockSpec((D, D), lambda i: (0, 0)),
            pl.BlockSpec((1, D), lambda i: (0, 0)),
        ],
        out_specs=pl.BlockSpec((EB, D), lambda i: (i, 0)),
        out_shape=jax.ShapeDtypeStruct((E, D), _f32),
    )(g, w0, b0, w1, b1)


def kernel(x, edge_index, edge_attr, batch, W_pre, b_pre,
           conv0_W0, conv0_b0, conv0_W1, conv0_b1, bn0_g, bn0_b,
           conv1_W0, conv1_b0, conv1_W1, conv1_b1, bn1_g, bn1_b,
           conv2_W0, conv2_b0, conv2_W1, conv2_b1, bn2_g, bn2_b,
           W_post, b_post, W_out, b_out):
    src = edge_index[0]
    dst = edge_index[1]
    ea0 = jnp.asarray(edge_attr[:, 0])
    ea1 = jnp.asarray(edge_attr[:, 1])
    batch2 = batch.reshape(1, N)

    convs = [(conv0_W0, conv0_b0, conv0_W1, conv0_b1, bn0_g, bn0_b),
             (conv1_W0, conv1_b0, conv1_W1, conv1_b1, bn1_g, bn1_b),
             (conv2_W0, conv2_b0, conv2_W1, conv2_b1, bn2_g, bn2_b)]

    out = _d0(x, W_pre, b_pre.reshape(1, D))
    degt = None
    acc = None
    for i, (W0, b0, W1, b1, g, bb) in enumerate(convs):
        gmat = _get_gather()(out, src, dst, ea0, ea1)
        u = _m(gmat, W0, b0.reshape(1, D), W1, b1.reshape(1, D))
        if i == 0:
            acc, degt = _get_scatter(True)(u, dst)
        else:
            acc = _get_scatter(False)(u, dst)
        if i < 2:
            out = _dmid(acc, degt, g.reshape(1, D), bb.reshape(1, D))
    o = _df(acc, degt, bn2_g.reshape(1, D), bn2_b.reshape(1, D), batch2,
            W_post, b_post.reshape(1, D), W_out, b_out.reshape(1, 1))
    return o.reshape(-1)


# speedup vs baseline: 1.2735x; 1.2100x over previous
import functools

import jax
import jax.numpy as jnp
from jax import lax
from jax.experimental import pallas as pl
from jax.experimental.pallas import tpu as pltpu
from jax.experimental.pallas import tpu_sc as plsc

N = 10000
E = 320000
F_IN = 128
D = 64
NG = 64
G3 = 3 * D

NC = 2
NS = 16
NW = NC * NS
EW = E // NW
CK = 80
NCHUNK = EW // CK
RA = 624
RL = 640
DEGW = 16

_f32 = jnp.float32
_HI = lax.Precision.HIGHEST


def _mesh():
    return plsc.VectorSubcoreMesh(core_axis_name="c", subcore_axis_name="s")


_CP = pltpu.CompilerParams(needs_layout_passes=False,
                           use_tc_tiling_on_sc=False)


def _gather_body(out_hbm, src_hbm, dst_hbm, ea0_hbm, ea1_hbm, g_hbm,
                 src_v, dst_v, ea0_v, ea1_v, xi_rows, xj_rows, g_buf,
                 sem_i, sem_j, sem_x):
    c = lax.axis_index("c")
    s = lax.axis_index("s")
    wid = s * NC + c
    wbase = wid * EW

    def chunk(i, carry):
        base = wbase + i * CK
        c1 = pltpu.async_copy(src_hbm.at[pl.ds(base, CK)], src_v, sem_i)
        c2 = pltpu.async_copy(dst_hbm.at[pl.ds(base, CK)], dst_v, sem_j)
        c3 = pltpu.async_copy(ea0_hbm.at[pl.ds(base, CK)], ea0_v, sem_x)
        c4 = pltpu.async_copy(ea1_hbm.at[pl.ds(base, CK)], ea1_v, sem_x)
        c1.wait()
        c2.wait()
        ci = pltpu.async_copy(out_hbm.at[dst_v], xi_rows, sem_i)
        cj = pltpu.async_copy(out_hbm.at[src_v], xj_rows, sem_j)
        c3.wait()
        c4.wait()
        ci.wait()
        cj.wait()

        def group(g, carry2):
            e0 = g * 16
            p0v = ea0_v[pl.ds(e0, 16)]
            p1v = ea1_v[pl.ds(e0, 16)]
            for lane in range(16):
                e = e0 + lane
                p0 = p0v[lane]
                p1 = p1v[lane]
                for j in range(D // 16):
                    xj = xj_rows[e, pl.ds(j * 16, 16)]
                    g_buf[e, pl.ds(j * 16, 16)] = xi_rows[e, pl.ds(j * 16, 16)]
                    g_buf[e, pl.ds(D + j * 16, 16)] = p0 * xj
                    g_buf[e, pl.ds(2 * D + j * 16, 16)] = p1 * xj
            return carry2

        lax.fori_loop(0, CK // 16, group, 0)
        pltpu.sync_copy(g_buf, g_hbm.at[pl.ds(base, CK)])
        return carry

    lax.fori_loop(0, NCHUNK, chunk, 0)


@functools.cache
def _get_gather():
    return pl.kernel(
        _gather_body,
        out_type=jax.ShapeDtypeStruct((E, G3), _f32),
        mesh=_mesh(),
        compiler_params=_CP,
        scratch_types=[
            pltpu.VMEM((CK,), jnp.int32),
            pltpu.VMEM((CK,), jnp.int32),
            pltpu.VMEM((CK,), _f32),
            pltpu.VMEM((CK,), _f32),
            pltpu.VMEM((CK, D), _f32),
            pltpu.VMEM((CK, D), _f32),
            pltpu.VMEM((CK, G3), _f32),
            pltpu.SemaphoreType.DMA,
            pltpu.SemaphoreType.DMA,
            pltpu.SemaphoreType.DMA,
        ],
    )


EH = E // 2
SCAN_B = 4000
NBATCH = EH // SCAN_B
CAP = 4016
GB = 80
RJ = RL + 16
JUNK = RL


def _make_scatter(with_deg):
    def body(u_hbm, dst_hbm, *rest):
        if with_deg:
            (acc_out, deg_out, acc_l, deg_l, dstb0, dstb1, prow_l, dloc_l,
             ur0, ur1, semd0, semd1, semu0, semu1) = rest
        else:
            (acc_out, acc_l, dstb0, dstb1, prow_l, dloc_l,
             ur0, ur1, semd0, semd1, semu0, semu1) = rest
        c = lax.axis_index("c")
        s = lax.axis_index("s")
        lo = s * RA
        mysz = jnp.where(s == NS - 1, RL, RA)
        hi = lo + mysz
        ebase = c * EH
        io16 = lax.iota(jnp.int32, 16)

        def zacc(r, cr):
            for j in range(D // 16):
                acc_l[r, pl.ds(j * 16, 16)] = jnp.zeros((16,), _f32)
            if with_deg:
                deg_l[r, pl.ds(0, 16)] = jnp.zeros((16,), _f32)
            return cr

        lax.fori_loop(0, RJ, zacc, 0)

        def zlist(q, cr):
            prow_l[pl.ds(q * 16, 16)] = jnp.zeros((16,), jnp.int32)
            return cr

        lax.fori_loop(0, CAP // 16, zlist, 0)

        def start_dst(bi, buf, sem):
            pltpu.async_copy(
                dst_hbm.at[pl.ds(ebase + bi * SCAN_B, SCAN_B)], buf, sem)

        def wait_dst(buf, sem):
            pltpu.make_async_copy(
                dst_hbm.at[pl.ds(ebase, SCAN_B)], buf, sem).wait()

        def start_u(b, ur, semu):
            pltpu.async_copy(u_hbm.at[prow_l.at[pl.ds(b * GB, GB)]],
                             ur, semu)

        def wait_u(ur, semu):
            pltpu.make_async_copy(u_hbm.at[pl.ds(0, GB)], ur, semu).wait()

        start_dst(0, dstb0, semd0)
        start_dst(1, dstb1, semd1)

        def half_batch(bi, dstb, semd):
            wait_dst(dstb, semd)

            def scan_g(g, off):
                dv = dstb[pl.ds(g * 16, 16)]
                m = (dv >= lo) & (dv < hi)
                csum = plsc.cumsum(m.astype(jnp.int32))
                idxs = off + csum - 1
                posv = ebase + bi * SCAN_B + g * 16 + io16
                plsc.store_scatter(prow_l, [idxs], posv, mask=m)
                plsc.store_scatter(dloc_l, [idxs], dv - lo, mask=m)
                return off + csum[15]

            k = lax.fori_loop(0, SCAN_B // 16, scan_g, 0,
                              unroll=4)

            @pl.when(bi + 2 < NBATCH)
            def _pf():
                start_dst(bi + 2, dstb, semd)

            padidx = k + io16
            plsc.store_scatter(dloc_l, [padidx],
                               jnp.full((16,), JUNK, jnp.int32))
            plsc.store_scatter(prow_l, [padidx], jnp.zeros((16,), jnp.int32))
            nb = (k + GB - 1) // GB

            @pl.when(nb > 0)
            def _s0():
                start_u(0, ur0, semu0)

            @pl.when(nb > 1)
            def _s1():
                start_u(1, ur1, semu1)

            def process(b, ur):
                kb = k - b * GB
                ng = jnp.minimum(GB // 16, (kb + 15) >> 4)

                def grp(gl, cr3):
                    qb = b * GB + gl * 16
                    rv = dloc_l[pl.ds(qb, 16)]
                    for lane in range(16):
                        r = rv[lane]
                        for j in range(D // 16):
                            acc_l[r, pl.ds(j * 16, 16)] = (
                                acc_l[r, pl.ds(j * 16, 16)]
                                + ur[gl * 16 + lane, pl.ds(j * 16, 16)])
                        if with_deg:
                            deg_l[r, pl.ds(0, 16)] = (
                                deg_l[r, pl.ds(0, 16)]
                                + jnp.ones((16,), _f32))
                    return cr3

                lax.fori_loop(0, ng, grp, 0)

            def bpair(bp, cr2):
                b0 = 2 * bp
                b1 = 2 * bp + 1

                @pl.when(b0 < nb)
                def _p0():
                    wait_u(ur0, semu0)

                    @pl.when(b0 + 2 < nb)
                    def _n0():
                        start_u(b0 + 2, ur0, semu0)

                    process(b0, ur0)

                @pl.when(b1 < nb)
                def _p1():
                    wait_u(ur1, semu1)

                    @pl.when(b1 + 2 < nb)
                    def _n1():
                        start_u(b1 + 2, ur1, semu1)

                    process(b1, ur1)
                return cr2

            lax.fori_loop(0, (nb + 1) >> 1, bpair, 0)

        def batch_pair(q, cr):
            half_batch(2 * q, dstb0, semd0)
            half_batch(2 * q + 1, dstb1, semd1)
            return cr

        lax.fori_loop(0, NBATCH // 2, batch_pair, 0)

        @pl.when(s < NS - 1)
        def _dump_small():
            pltpu.sync_copy(acc_l.at[pl.ds(0, RA)],
                            acc_out.at[c, pl.ds(lo, RA)])
            if with_deg:
                pltpu.sync_copy(deg_l.at[pl.ds(0, RA)],
                                deg_out.at[c, pl.ds(lo, RA)])

        @pl.when(s == NS - 1)
        def _dump_last():
            pltpu.sync_copy(acc_l.at[pl.ds(0, RL)],
                            acc_out.at[c, pl.ds((NS - 1) * RA, RL)])
            if with_deg:
                pltpu.sync_copy(deg_l.at[pl.ds(0, RL)],
                                deg_out.at[c, pl.ds((NS - 1) * RA, RL)])

    scratch = [
        pltpu.VMEM((RJ, D), _f32),
        pltpu.VMEM((RJ, DEGW), _f32),
        pltpu.VMEM((SCAN_B,), jnp.int32),
        pltpu.VMEM((SCAN_B,), jnp.int32),
        pltpu.VMEM((CAP,), jnp.int32),
        pltpu.VMEM((CAP,), jnp.int32),
        pltpu.VMEM((GB, D), _f32),
        pltpu.VMEM((GB, D), _f32),
        pltpu.SemaphoreType.DMA,
        pltpu.SemaphoreType.DMA,
        pltpu.SemaphoreType.DMA,
        pltpu.SemaphoreType.DMA,
    ]
    if with_deg:
        out_type = (jax.ShapeDtypeStruct((NC, N, D), _f32),
                    jax.ShapeDtypeStruct((NC, N, DEGW), _f32))
    else:
        out_type = jax.ShapeDtypeStruct((NC, N, D), _f32)
        scratch = scratch[:1] + scratch[2:]
    return pl.kernel(
        body, out_type=out_type, mesh=_mesh(),
        compiler_params=_CP,
        scratch_types=scratch)


_get_scatter = functools.cache(_make_scatter)


def _d0_body(x_ref, wpre_ref, bpre_ref, out_ref):
    out_ref[...] = jnp.maximum(x_ref[...] @ wpre_ref[...] + bpre_ref[...],
                               0.0)


def _d0(x, wpre, bpre):
    return pl.pallas_call(
        _d0_body,
        out_shape=jax.ShapeDtypeStruct((N, D), _f32),
    )(x, wpre, bpre)


def _norm_agg(acc_ref, deg_ref, g_ref, bb_ref):
    deg = jnp.maximum(deg_ref[0, :, 0:1] + deg_ref[1, :, 0:1], 1.0)
    agg = (acc_ref[0] + acc_ref[1]) / deg
    mu = jnp.mean(agg, axis=0, keepdims=True)
    xc = agg - mu
    var = jnp.mean(xc * xc, axis=0, keepdims=True)
    return g_ref[...] * xc * lax.rsqrt(var + 1e-5) + bb_ref[...]


def _dmid_body(acc_ref, deg_ref, g_ref, bb_ref, out_ref):
    out_ref[...] = _norm_agg(acc_ref, deg_ref, g_ref, bb_ref)


def _dmid(acc, degt, g, bb):
    return pl.pallas_call(
        _dmid_body,
        out_shape=jax.ShapeDtypeStruct((N, D), _f32),
    )(acc, degt, g, bb)


def _df_body(acc_ref, deg_ref, g_ref, bb_ref, batch_ref, wpost_ref, bpost_ref,
             wout_ref, bout_ref, o_ref):
    outn = _norm_agg(acc_ref, deg_ref, g_ref, bb_ref)
    ids = lax.broadcasted_iota(jnp.int32, (NG, N), 0)
    oh = (batch_ref[...] == ids).astype(_f32)
    psum = jnp.dot(oh, outn, precision=_HI)
    cnt = jnp.maximum(jnp.sum(oh, axis=1, keepdims=True), 1.0)
    pooled = psum / cnt
    h = jnp.maximum(pooled @ wpost_ref[...] + bpost_ref[...], 0.0)
    o_ref[...] = h @ wout_ref[...] + bout_ref[...]


def _df(acc, degt, g, bb, batch2, wpost, bpost, wout, bout):
    return pl.pallas_call(
        _df_body,
        out_shape=jax.ShapeDtypeStruct((NG, 1), _f32),
    )(acc, degt, g, bb, batch2, wpost, bpost, wout, bout)


EB = 2000


def _m_body(g_ref, w0_ref, b0_ref, w1_ref, b1_ref, u_ref):
    m = jnp.maximum(g_ref[...] @ w0_ref[...] + b0_ref[...], 0.0)
    u_ref[...] = jnp.maximum(m @ w1_ref[...] + b1_ref[...], 0.0)


def _m(g, w0, b0, w1, b1):
    return pl.pallas_call(
        _m_body,
        grid=(E // EB,),
        in_specs=[
            pl.BlockSpec((EB, G3), lambda i: (i, 0)),
            pl.BlockSpec((G3, D), lambda i: (0, 0)),
            pl.BlockSpec((1, D), lambda i: (0, 0)),
            pl.BlockSpec((D, D), lambda i: (0, 0)),
            pl.BlockSpec((1, D), lambda i: (0, 0)),
        ],
        out_specs=pl.BlockSpec((EB, D), lambda i: (i, 0)),
        out_shape=jax.ShapeDtypeStruct((E, D), _f32),
    )(g, w0, b0, w1, b1)


def kernel(x, edge_index, edge_attr, batch, W_pre, b_pre,
           conv0_W0, conv0_b0, conv0_W1, conv0_b1, bn0_g, bn0_b,
           conv1_W0, conv1_b0, conv1_W1, conv1_b1, bn1_g, bn1_b,
           conv2_W0, conv2_b0, conv2_W1, conv2_b1, bn2_g, bn2_b,
           W_post, b_post, W_out, b_out):
    src = edge_index[0]
    dst = edge_index[1]
    ea0 = jnp.asarray(edge_attr[:, 0])
    ea1 = jnp.asarray(edge_attr[:, 1])
    batch2 = batch.reshape(1, N)

    convs = [(conv0_W0, conv0_b0, conv0_W1, conv0_b1, bn0_g, bn0_b),
             (conv1_W0, conv1_b0, conv1_W1, conv1_b1, bn1_g, bn1_b),
             (conv2_W0, conv2_b0, conv2_W1, conv2_b1, bn2_g, bn2_b)]

    out = _d0(x, W_pre, b_pre.reshape(1, D))
    degt = None
    acc = None
    for i, (W0, b0, W1, b1, g, bb) in enumerate(convs):
        gmat = _get_gather()(out, src, dst, ea0, ea1)
        u = _m(gmat, W0, b0.reshape(1, D), W1, b1.reshape(1, D))
        if i == 0:
            acc, degt = _get_scatter(True)(u, dst)
        else:
            acc = _get_scatter(False)(u, dst)
        if i < 2:
            out = _dmid(acc, degt, g.reshape(1, D), bb.reshape(1, D))
    o = _df(acc, degt, bn2_g.reshape(1, D), bn2_b.reshape(1, D), batch2,
            W_post, b_post.reshape(1, D), W_out, b_out.reshape(1, 1))
    return o.reshape(-1)
